# Initial kernel scaffold; baseline (speedup 1.0000x reference)
import jax
import jax.numpy as jnp
from jax.experimental import pallas as pl


def kernel(x, edge_index, W1_l, b1_l, W1_r, W2_l, b2_l, W2_r, Wc, bc):
    raise NotImplementedError("write your pallas kernel here")


# speedup vs baseline: 6.8078x; 6.8078x over previous
import functools

import jax
import jax.numpy as jnp
from jax import lax
from jax.experimental import pallas as pl
from jax.experimental.pallas import tpu as pltpu
from jax.experimental.pallas import tpu_sc as plsc

NC = 2
NS = 16
NW = NC * NS
CH = 128


def _sc_agg_rows_body(n_pad, d, cpb, xp, srcm, dstm, z2d, z1d,
                      out_s, out_c0, out_c1,
                      idxd_v, sba, sbb, bufa, bufb, ones_v,
                      semsa, semsb, semga, semgb,
                      acc, cnt):
    cid = lax.axis_index("c")
    sid = lax.axis_index("s")
    w = cid * NS + sid
    rpt = n_pad // NS
    rows0 = sid * rpt

    pltpu.sync_copy(z2d.at[pl.ds(rows0, rpt)], acc.at[pl.ds(rows0, rpt)])
    pltpu.sync_copy(z1d.at[pl.ds(rows0, rpt)], cnt.at[pl.ds(rows0, rpt)])

    def _set_ones(k, carry):
        ones_v[pl.ds(k * 16, 16)] = jnp.ones((16,), jnp.float32)
        return carry
    lax.fori_loop(0, CH // 16, _set_ones, 0)

    c0 = w * cpb
    pltpu.sync_copy(dstm.at[pl.ds(c0, cpb)], idxd_v)
    plsc.subcore_barrier()

    pltpu.async_copy(srcm.at[c0], sba, semsa)
    pltpu.async_copy(srcm.at[c0 + 1], sbb, semsb)
    pltpu.make_async_copy(srcm.at[c0], sba, semsa).wait()
    pltpu.async_copy(xp.at[sba], bufa, semga)

    def _step(j, carry):
        ia = 2 * j
        ib = 2 * j + 1
        pltpu.make_async_copy(srcm.at[c0 + ib], sbb, semsb).wait()
        pltpu.async_copy(xp.at[sbb], bufb, semgb)
        pltpu.make_async_copy(xp.at[sba], bufa, semga).wait()

        @pl.when(ia + 2 < cpb)
        def _():
            pltpu.async_copy(srcm.at[c0 + ia + 2], sba, semsa)

        pltpu.sync_copy(bufa, acc.at[idxd_v.at[ia]], add=True)
        pltpu.sync_copy(ones_v, cnt.at[idxd_v.at[ia]], add=True)

        @pl.when(ia + 2 < cpb)
        def _():
            pltpu.make_async_copy(srcm.at[c0 + ia + 2], sba, semsa).wait()
            pltpu.async_copy(xp.at[sba], bufa, semga)

        pltpu.make_async_copy(xp.at[sbb], bufb, semgb).wait()

        @pl.when(ib + 2 < cpb)
        def _():
            pltpu.async_copy(srcm.at[c0 + ib + 2], sbb, semsb)

        pltpu.sync_copy(bufb, acc.at[idxd_v.at[ib]], add=True)
        pltpu.sync_copy(ones_v, cnt.at[idxd_v.at[ib]], add=True)
        return carry

    lax.fori_loop(0, cpb // 2, _step, 0)
    plsc.subcore_barrier()

    pltpu.sync_copy(acc.at[pl.ds(rows0, rpt)], out_s.at[cid, pl.ds(rows0, rpt)])

    @pl.when(cid == 0)
    def _():
        pltpu.sync_copy(cnt.at[pl.ds(rows0, rpt)], out_c0.at[pl.ds(rows0, rpt)])

    @pl.when(cid == 1)
    def _():
        pltpu.sync_copy(cnt.at[pl.ds(rows0, rpt)], out_c1.at[pl.ds(rows0, rpt)])


def _sc_agg_scalar_body(n_pad, cpd, u, srcm, dstm, z1d, cntm, vb, out,
                        idxd_v, sba, sbb, bufa, bufb, ab, cb, vbuf, ob,
                        semsa, semsb, semga, semgb, acc):
    cid = lax.axis_index("c")
    sid = lax.axis_index("s")
    rpt = n_pad // NS
    rows0 = sid * rpt

    @pl.when(cid == 0)
    def _():
        pltpu.sync_copy(z1d.at[pl.ds(rows0, rpt)], acc.at[pl.ds(rows0, rpt)])
        c0 = sid * cpd
        pltpu.sync_copy(dstm.at[pl.ds(c0, cpd)], idxd_v)

    plsc.subcore_barrier()

    @pl.when(cid == 0)
    def _():
        c0 = sid * cpd
        pltpu.async_copy(srcm.at[c0], sba, semsa)
        pltpu.async_copy(srcm.at[c0 + 1], sbb, semsb)
        pltpu.make_async_copy(srcm.at[c0], sba, semsa).wait()
        pltpu.async_copy(u.at[sba], bufa, semga)

        def _step(j, carry):
            ia = 2 * j
            ib = 2 * j + 1
            pltpu.make_async_copy(srcm.at[c0 + ib], sbb, semsb).wait()
            pltpu.async_copy(u.at[sbb], bufb, semgb)
            pltpu.make_async_copy(u.at[sba], bufa, semga).wait()

            @pl.when(ia + 2 < cpd)
            def _():
                pltpu.async_copy(srcm.at[c0 + ia + 2], sba, semsa)

            pltpu.sync_copy(bufa, acc.at[idxd_v.at[ia]], add=True)

            @pl.when(ia + 2 < cpd)
            def _():
                pltpu.make_async_copy(srcm.at[c0 + ia + 2], sba, semsa).wait()
                pltpu.async_copy(u.at[sba], bufa, semga)

            pltpu.make_async_copy(u.at[sbb], bufb, semgb).wait()

            @pl.when(ib + 2 < cpd)
            def _():
                pltpu.async_copy(srcm.at[c0 + ib + 2], sbb, semsb)

            pltpu.sync_copy(bufb, acc.at[idxd_v.at[ib]], add=True)
            return carry

        lax.fori_loop(0, cpd // 2, _step, 0)

    plsc.subcore_barrier()

    @pl.when(cid == 0)
    def _():
        pltpu.sync_copy(acc.at[pl.ds(rows0, rpt)], ab)
        pltpu.sync_copy(cntm.at[pl.ds(rows0, rpt)], cb)
        pltpu.sync_copy(vb.at[pl.ds(rows0, rpt)], vbuf)

        def _fin(k, carry):
            s = ab[pl.ds(k * 16, 16)]
            c = cb[pl.ds(k * 16, 16)]
            v = vbuf[pl.ds(k * 16, 16)]
            ob[pl.ds(k * 16, 16)] = s / c + v
            return carry

        lax.fori_loop(0, rpt // 16, _fin, 0)
        pltpu.sync_copy(ob, out.at[pl.ds(rows0, rpt)])


def _tc_dense_body(s2, c3, xb, w1l, w1r, b1, w2l, w2r, wc, b2, bcb,
                   u_o, v_o, cm_o):
    s = s2[0] + s2[1]
    cnt = c3[0] + c3[1]
    cntm = jnp.maximum(cnt, 1.0)
    mean = s / cntm
    h = jnp.dot(mean, w1l[...], preferred_element_type=jnp.float32)
    h = h + jnp.dot(xb[...], w1r[...], preferred_element_type=jnp.float32)
    h = jnp.maximum(h + b1[...], 0.0)
    w2 = jnp.dot(w2l[...], wc[...], preferred_element_type=jnp.float32)
    wr = jnp.dot(w2r[...], wc[...], preferred_element_type=jnp.float32)
    c0s = jnp.dot(b2[...], wc[...], preferred_element_type=jnp.float32)
    u_o[...] = jnp.dot(h, w2, preferred_element_type=jnp.float32)
    v_o[...] = jnp.dot(h, wr, preferred_element_type=jnp.float32) + c0s + bcb[...]
    cm_o[...] = cntm


def kernel(x, edge_index, W1_l, b1_l, W1_r, W2_l, b2_l, W2_r, Wc, bc):
    n, d = x.shape
    e = edge_index.shape[1]
    n_pad = (-(-n // CH) + 1) * CH
    n_pad = -(-n_pad // (NS * CH)) * NS * CH
    cpb = -(-e // (NW * CH))
    cpb = ((cpb + 7) // 8) * 8
    e_pad = NW * cpb * CH
    cpd = cpb * NC

    f32 = jnp.float32
    xp = jnp.pad(x, ((0, n_pad - n), (0, 0)))
    pad = jnp.full((e_pad - e,), n, jnp.int32)
    srcm = jnp.concatenate([edge_index[0], pad]).reshape(e_pad // CH, CH)
    dstm = jnp.concatenate([edge_index[1], pad]).reshape(e_pad // CH, CH)
    z2d = jnp.zeros((n_pad, d), f32)
    z1d = jnp.zeros((n_pad,), f32)

    mesh = plsc.VectorSubcoreMesh(core_axis_name="c", subcore_axis_name="s")

    agg_rows = pl.kernel(
        functools.partial(_sc_agg_rows_body, n_pad, d, cpb),
        out_type=[jax.ShapeDtypeStruct((NC, n_pad, d), f32),
                  jax.ShapeDtypeStruct((n_pad,), f32),
                  jax.ShapeDtypeStruct((n_pad,), f32)],
        mesh=mesh,
        scratch_types=[
            pltpu.VMEM((cpb, CH), jnp.int32),
            pltpu.VMEM((CH,), jnp.int32),
            pltpu.VMEM((CH,), jnp.int32),
            pltpu.VMEM((CH, d), f32),
            pltpu.VMEM((CH, d), f32),
            pltpu.VMEM((CH,), f32),
            pltpu.SemaphoreType.DMA,
            pltpu.SemaphoreType.DMA,
            pltpu.SemaphoreType.DMA,
            pltpu.SemaphoreType.DMA,
            pltpu.VMEM_SHARED((n_pad, d), f32),
            pltpu.VMEM_SHARED((n_pad,), f32),
        ],
    )
    s_part, c_part0, c_part1 = agg_rows(xp, srcm, dstm, z2d, z1d)
    c_part = jnp.stack([c_part0, c_part1]).reshape(NC, n_pad, 1)

    grid_r = 1024
    gsteps = n_pad // grid_r
    u, v, cm = pl.pallas_call(
        _tc_dense_body,
        grid=(gsteps,),
        in_specs=[
            pl.BlockSpec((NC, grid_r, d), lambda i: (0, i, 0)),
            pl.BlockSpec((NC, grid_r, 1), lambda i: (0, i, 0)),
            pl.BlockSpec((grid_r, d), lambda i: (i, 0)),
            pl.BlockSpec((d, d), lambda i: (0, 0)),
            pl.BlockSpec((d, d), lambda i: (0, 0)),
            pl.BlockSpec((1, d), lambda i: (0, 0)),
            pl.BlockSpec((d, d), lambda i: (0, 0)),
            pl.BlockSpec((d, d), lambda i: (0, 0)),
            pl.BlockSpec((d, 1), lambda i: (0, 0)),
            pl.BlockSpec((1, d), lambda i: (0, 0)),
            pl.BlockSpec((1, 1), lambda i: (0, 0)),
        ],
        out_specs=[
            pl.BlockSpec((grid_r, 1), lambda i: (i, 0)),
            pl.BlockSpec((grid_r, 1), lambda i: (i, 0)),
            pl.BlockSpec((grid_r, 1), lambda i: (i, 0)),
        ],
        out_shape=[
            jax.ShapeDtypeStruct((n_pad, 1), f32),
            jax.ShapeDtypeStruct((n_pad, 1), f32),
            jax.ShapeDtypeStruct((n_pad, 1), f32),
        ],
        compiler_params=pltpu.CompilerParams(
            dimension_semantics=("arbitrary",)),
    )(s_part, c_part, xp, W1_l, W1_r,
      b1_l.reshape(1, d), W2_l, W2_r, Wc, b2_l.reshape(1, d),
      bc.reshape(1, 1))

    agg_scalar = pl.kernel(
        functools.partial(_sc_agg_scalar_body, n_pad, cpd),
        out_type=jax.ShapeDtypeStruct((n_pad,), f32),
        mesh=mesh,
        scratch_types=[
            pltpu.VMEM((cpd, CH), jnp.int32),
            pltpu.VMEM((CH,), jnp.int32),
            pltpu.VMEM((CH,), jnp.int32),
            pltpu.VMEM((CH,), f32),
            pltpu.VMEM((CH,), f32),
            pltpu.VMEM((n_pad // NS,), f32),
            pltpu.VMEM((n_pad // NS,), f32),
            pltpu.VMEM((n_pad // NS,), f32),
            pltpu.VMEM((n_pad // NS,), f32),
            pltpu.SemaphoreType.DMA,
            pltpu.SemaphoreType.DMA,
            pltpu.SemaphoreType.DMA,
            pltpu.SemaphoreType.DMA,
            pltpu.VMEM_SHARED((n_pad,), f32),
        ],
    )
    out_full = agg_scalar(u.reshape(n_pad), srcm, dstm, z1d,
                          cm.reshape(n_pad), v.reshape(n_pad))
    return out_full[:n]
